# baseline (speedup 1.0000x reference)
import functools

import jax
import jax.numpy as jnp
from jax import lax
from jax.experimental import pallas as pl
from jax.experimental.pallas import tpu as pltpu
from jax.experimental.pallas import tpu_sc as plsc

N_NODES = 10000
HIDDEN = 128
N_EDGES = 320000
NUM_CONVS = 3

NC = 2
NS = 16
NW = NC * NS
E_PER_W = N_EDGES // NW
CHUNK = 40
NCHUNK = E_PER_W // CHUNK
NPAIR = NCHUNK // 2
RCHUNK = 400
NRCHUNK = N_NODES // RCHUNK
HV = HIDDEN // 16


def _sc_agg_kernel(x_hbm, ea_hbm, src_hbm, dst_hbm, out_hbm,
                   sidx, didx, ea_v, row_v, sc_v, agg_sh,
                   isem0, isem1, isem2, gsem0, gsem1, gsem2,
                   easem0, easem1, dsem0, dsem1, ssem0, ssem1):
    c = lax.axis_index("c")
    s = lax.axis_index("s")
    wid = c * NS + s
    base_ch = wid * NCHUNK
    isem = (isem0, isem1, isem2)
    gsem = (gsem0, gsem1, gsem2)
    easem = (easem0, easem1)
    dsem = (dsem0, dsem1)
    ssem = (ssem0, ssem1)

    def zbody(i, _):
        for k in range(HV):
            row_v[0, i, pl.ds(k * 16, 16)] = jnp.zeros((16,), jnp.float32)
        return 0
    lax.fori_loop(0, CHUNK, zbody, 0)

    def zcopy(r, _):
        idx = s + r * NS

        @pl.when(idx < NCHUNK)
        def _():
            pltpu.sync_copy(row_v.at[0], agg_sh.at[pl.ds(idx * CHUNK, CHUNK)])
        return 0
    lax.fori_loop(0, (NCHUNK + NS - 1) // NS, zcopy, 0)

    plsc.subcore_barrier()

    def _issue_gather(j, r):
        pltpu.async_copy(x_hbm.at[sidx.at[r, 0]], row_v.at[r], gsem[r])

    def _issue_ea(j, b):
        pltpu.async_copy(ea_hbm.at[pl.ds(wid * E_PER_W + j * CHUNK, CHUNK)],
                         ea_v.at[b], easem[b])

    for r in range(3):
        pltpu.async_copy(src_hbm.at[base_ch + r], sidx.at[r], isem[r])
    for r in range(2):
        pltpu.make_async_copy(src_hbm.at[0], sidx.at[r], isem[r]).wait()
        _issue_gather(r, r)
    _issue_ea(0, 0)

    def process(j, u):
        r = u % 3
        b = u % 2

        @pl.when(j < NCHUNK)
        def _():
            pltpu.make_async_copy(x_hbm.at[pl.ds(0, CHUNK)], row_v.at[r],
                                  gsem[r]).wait()
            pltpu.make_async_copy(ea_hbm.at[pl.ds(0, CHUNK)], ea_v.at[b],
                                  easem[b]).wait()

            @pl.when(j + 3 < NCHUNK)
            def _():
                pltpu.async_copy(src_hbm.at[base_ch + j + 3], sidx.at[r],
                                 isem[r])

            @pl.when(j + 2 < NCHUNK)
            def _():
                pltpu.make_async_copy(src_hbm.at[0], sidx.at[(u + 2) % 3],
                                      isem[(u + 2) % 3]).wait()
                _issue_gather(j + 2, (u + 2) % 3)

            @pl.when(j + 1 < NCHUNK)
            def _():
                _issue_ea(j + 1, (u + 1) % 2)

            @pl.when(j >= 2)
            def _():
                pltpu.make_async_copy(sc_v.at[b], agg_sh.at[pl.ds(0, CHUNK)],
                                      ssem[b]).wait()

            pltpu.async_copy(dst_hbm.at[base_ch + j], didx.at[b], dsem[b])

            def rbody(ii, _):
                for rr in range(2):
                    i = 2 * ii + rr
                    for k in range(HV):
                        a = row_v[r, i, pl.ds(k * 16, 16)]
                        e = ea_v[b, i, pl.ds(k * 16, 16)]
                        sc_v[b, i, pl.ds(k * 16, 16)] = jnp.maximum(a + e, 0.0)
                return 0
            lax.fori_loop(0, CHUNK // 2, rbody, 0)

            pltpu.make_async_copy(dst_hbm.at[0], didx.at[b], dsem[b]).wait()
            pltpu.async_copy(sc_v.at[b], agg_sh.at[didx.at[b, 0]], ssem[b],
                             add=True)

    def body(t, _):
        for u in range(6):
            process(6 * t + u, u)
        return 0
    lax.fori_loop(0, (NCHUNK + 5) // 6, body, 0)

    for b in range(2):
        pltpu.make_async_copy(sc_v.at[b], agg_sh.at[pl.ds(0, CHUNK)],
                              ssem[b]).wait()

    plsc.subcore_barrier()

    def ocopy(r, _):
        idx = s + r * NS

        @pl.when(idx < NRCHUNK)
        def _():
            pltpu.sync_copy(agg_sh.at[pl.ds(idx * RCHUNK, RCHUNK)],
                            out_hbm.at[c, pl.ds(idx * RCHUNK, RCHUNK)])
        return 0
    lax.fori_loop(0, (NRCHUNK + NS - 1) // NS, ocopy, 0)


@jax.jit
def _sc_agg(x, edge_attr, src3, dst3):
    mesh = plsc.VectorSubcoreMesh(core_axis_name="c", subcore_axis_name="s")
    return pl.kernel(
        _sc_agg_kernel,
        out_type=jax.ShapeDtypeStruct((NC, N_NODES, HIDDEN), jnp.float32),
        mesh=mesh,
        scratch_types=[
            pltpu.VMEM((3, 1, CHUNK), jnp.int32),
            pltpu.VMEM((2, 1, CHUNK), jnp.int32),
            pltpu.VMEM((2, CHUNK, HIDDEN), jnp.float32),
            pltpu.VMEM((3, CHUNK, HIDDEN), jnp.float32),
            pltpu.VMEM((2, CHUNK, HIDDEN), jnp.float32),
            pltpu.VMEM_SHARED((N_NODES, HIDDEN), jnp.float32),
        ] + [pltpu.SemaphoreType.DMA] * 12,
    )(x, edge_attr, src3, dst3)


def _tc_emb_kernel(z_ref, emb_ref, x_ref):
    z = z_ref[...]
    cols = lax.broadcasted_iota(jnp.int32, (N_NODES, 100), 1)
    oh = jnp.where(z == cols, 1.0, 0.0).astype(jnp.float32)
    x_ref[...] = jnp.dot(oh, emb_ref[...], preferred_element_type=jnp.float32,
                         precision=lax.Precision.HIGHEST)


@jax.jit
def _tc_emb(z, emb):
    return pl.pallas_call(
        _tc_emb_kernel,
        out_shape=jax.ShapeDtypeStruct((N_NODES, HIDDEN), jnp.float32),
    )(z.reshape(N_NODES, 1), emb)


def _tc_layer_kernel(agg_ref, x_ref, w1_ref, b1_ref, w2_ref, b2_ref,
                     g_ref, be_ref, o_ref, *, last):
    x = x_ref[...]
    out = agg_ref[0] + agg_ref[1] + x
    h = jnp.maximum(jnp.dot(out, w1_ref[...],
                            preferred_element_type=jnp.float32) + b1_ref[...], 0.0)
    y = jnp.dot(h, w2_ref[...], preferred_element_type=jnp.float32) + b2_ref[...]
    m = jnp.mean(y, axis=0, keepdims=True)
    d = y - m
    v = jnp.mean(d * d, axis=0, keepdims=True)
    y = g_ref[...] * d / jnp.sqrt(v + 1e-5) + be_ref[...]
    if not last:
        y = jnp.maximum(y, 0.0)
    o_ref[...] = y + x


@functools.partial(jax.jit, static_argnames=("last",))
def _tc_layer(agg2, x, p, last):
    return pl.pallas_call(
        functools.partial(_tc_layer_kernel, last=last),
        out_shape=jax.ShapeDtypeStruct((N_NODES, HIDDEN), jnp.float32),
    )(agg2, x, p["W1"], p["b1"].reshape(1, HIDDEN), p["W2"],
      p["b2"].reshape(1, HIDDEN), p["gamma"].reshape(1, HIDDEN),
      p["beta"].reshape(1, HIDDEN))


def kernel(z, edge_index, edge_attr, params):
    src3 = edge_index[0].astype(jnp.int32).reshape(NW * NCHUNK, 1, CHUNK)
    dst3 = edge_index[1].astype(jnp.int32).reshape(NW * NCHUNK, 1, CHUNK)
    x = _tc_emb(z.astype(jnp.int32), params["emb"])
    for i, p in enumerate(params["layers"]):
        agg2 = _sc_agg(x, edge_attr, src3, dst3)
        x = _tc_layer(agg2, x, p, last=(i == NUM_CONVS - 1))
    return x


# speedup vs baseline: 1.0786x; 1.0786x over previous
import functools

import jax
import jax.numpy as jnp
from jax import lax
from jax.experimental import pallas as pl
from jax.experimental.pallas import tpu as pltpu
from jax.experimental.pallas import tpu_sc as plsc

N_NODES = 10000
HIDDEN = 128
N_EDGES = 320000
NUM_CONVS = 3

NC = 2
NS = 16
NW = NC * NS
E_PER_W = N_EDGES // NW
CHUNK = 40
NCHUNK = E_PER_W // CHUNK
NPAIR = NCHUNK // 2
RCHUNK = 400
NRCHUNK = N_NODES // RCHUNK
HV = HIDDEN // 16


def _sc_agg_kernel(x_hbm, ea_hbm, src_hbm, dst_hbm, out_hbm,
                   sidx, didx, ea_v, row_v, sc_v, agg_sh,
                   isem0, isem1, dsem0, dsem1, gesem0, gesem1, ssem0, ssem1):
    c = lax.axis_index("c")
    s = lax.axis_index("s")
    wid = c * NS + s
    base_ch = wid * NCHUNK
    isem = (isem0, isem1)
    dsem = (dsem0, dsem1)
    gesem = (gesem0, gesem1)
    ssem = (ssem0, ssem1)

    def zbody(i, _):
        for k in range(HV):
            row_v[0, i, pl.ds(k * 16, 16)] = jnp.zeros((16,), jnp.float32)
        return 0
    lax.fori_loop(0, CHUNK, zbody, 0)

    def zcopy(r, _):
        idx = s + r * NS

        @pl.when(idx < NCHUNK)
        def _():
            pltpu.sync_copy(row_v.at[0], agg_sh.at[pl.ds(idx * CHUNK, CHUNK)])
        return 0
    lax.fori_loop(0, (NCHUNK + NS - 1) // NS, zcopy, 0)

    plsc.subcore_barrier()

    def _issue_gather(j, b):
        pltpu.async_copy(x_hbm.at[sidx.at[b, 0]], row_v.at[b], gesem[b])
        pltpu.async_copy(ea_hbm.at[pl.ds(wid * E_PER_W + j * CHUNK, CHUNK)],
                         ea_v.at[b], gesem[b])

    def _wait_ge(b):
        pltpu.make_async_copy(x_hbm.at[pl.ds(0, CHUNK)], row_v.at[b],
                              gesem[b]).wait()
        pltpu.make_async_copy(ea_hbm.at[pl.ds(0, CHUNK)], ea_v.at[b],
                              gesem[b]).wait()

    pltpu.async_copy(src_hbm.at[base_ch], sidx.at[0], isem0)
    pltpu.async_copy(src_hbm.at[base_ch + 1], sidx.at[1], isem1)
    pltpu.make_async_copy(src_hbm.at[0], sidx.at[0], isem0).wait()
    _issue_gather(0, 0)

    def body(t, _):
        for b in range(2):
            j = 2 * t + b

            if b == 0:
                pltpu.make_async_copy(src_hbm.at[0], sidx.at[1],
                                      isem[1]).wait()
                _issue_gather(j + 1, 1)
            else:
                @pl.when(t < NPAIR - 1)
                def _():
                    pltpu.make_async_copy(src_hbm.at[0], sidx.at[0],
                                          isem[0]).wait()
                    _issue_gather(j + 1, 0)

            _wait_ge(b)

            @pl.when(t >= 1)
            def _():
                pltpu.make_async_copy(sc_v.at[b], agg_sh.at[pl.ds(0, CHUNK)],
                                      ssem[b]).wait()

            @pl.when(t < NPAIR - 1)
            def _():
                pltpu.async_copy(src_hbm.at[base_ch + j + 2], sidx.at[b],
                                 isem[b])

            pltpu.async_copy(dst_hbm.at[base_ch + j], didx.at[b], dsem[b])

            def rbody(ii, _):
                for rr in range(2):
                    i = 2 * ii + rr
                    for k in range(HV):
                        a = row_v[b, i, pl.ds(k * 16, 16)]
                        e = ea_v[b, i, pl.ds(k * 16, 16)]
                        sc_v[b, i, pl.ds(k * 16, 16)] = jnp.maximum(a + e, 0.0)
                return 0
            lax.fori_loop(0, CHUNK // 2, rbody, 0)

            pltpu.make_async_copy(dst_hbm.at[0], didx.at[b], dsem[b]).wait()
            pltpu.async_copy(sc_v.at[b], agg_sh.at[didx.at[b, 0]], ssem[b],
                             add=True)
        return 0
    lax.fori_loop(0, NPAIR, body, 0)

    for b in range(2):
        pltpu.make_async_copy(sc_v.at[b], agg_sh.at[pl.ds(0, CHUNK)],
                              ssem[b]).wait()

    plsc.subcore_barrier()

    def ocopy(r, _):
        idx = s + r * NS

        @pl.when(idx < NRCHUNK)
        def _():
            pltpu.sync_copy(agg_sh.at[pl.ds(idx * RCHUNK, RCHUNK)],
                            out_hbm.at[c, pl.ds(idx * RCHUNK, RCHUNK)])
        return 0
    lax.fori_loop(0, (NRCHUNK + NS - 1) // NS, ocopy, 0)


@jax.jit
def _sc_agg(x, edge_attr, src3, dst3):
    mesh = plsc.VectorSubcoreMesh(core_axis_name="c", subcore_axis_name="s")
    return pl.kernel(
        _sc_agg_kernel,
        out_type=jax.ShapeDtypeStruct((NC, N_NODES, HIDDEN), jnp.float32),
        mesh=mesh,
        scratch_types=[
            pltpu.VMEM((2, 1, CHUNK), jnp.int32),
            pltpu.VMEM((2, 1, CHUNK), jnp.int32),
            pltpu.VMEM((2, CHUNK, HIDDEN), jnp.float32),
            pltpu.VMEM((2, CHUNK, HIDDEN), jnp.float32),
            pltpu.VMEM((2, CHUNK, HIDDEN), jnp.float32),
            pltpu.VMEM_SHARED((N_NODES, HIDDEN), jnp.float32),
            pltpu.SemaphoreType.DMA,
            pltpu.SemaphoreType.DMA,
            pltpu.SemaphoreType.DMA,
            pltpu.SemaphoreType.DMA,
            pltpu.SemaphoreType.DMA,
            pltpu.SemaphoreType.DMA,
            pltpu.SemaphoreType.DMA,
            pltpu.SemaphoreType.DMA,
        ],
    )(x, edge_attr, src3, dst3)


def _tc_emb_kernel(z_ref, emb_ref, x_ref):
    z = z_ref[...]
    cols = lax.broadcasted_iota(jnp.int32, (N_NODES, 100), 1)
    oh = jnp.where(z == cols, 1.0, 0.0).astype(jnp.float32)
    x_ref[...] = jnp.dot(oh, emb_ref[...], preferred_element_type=jnp.float32,
                         precision=lax.Precision.HIGHEST)


@jax.jit
def _tc_emb(z, emb):
    return pl.pallas_call(
        _tc_emb_kernel,
        out_shape=jax.ShapeDtypeStruct((N_NODES, HIDDEN), jnp.float32),
    )(z.reshape(N_NODES, 1), emb)


def _tc_layer_kernel(agg_ref, x_ref, w1_ref, b1_ref, w2_ref, b2_ref,
                     g_ref, be_ref, o_ref, *, last):
    x = x_ref[...]
    out = agg_ref[0] + agg_ref[1] + x
    h = jnp.maximum(jnp.dot(out, w1_ref[...],
                            preferred_element_type=jnp.float32) + b1_ref[...], 0.0)
    y = jnp.dot(h, w2_ref[...], preferred_element_type=jnp.float32) + b2_ref[...]
    m = jnp.mean(y, axis=0, keepdims=True)
    d = y - m
    v = jnp.mean(d * d, axis=0, keepdims=True)
    y = g_ref[...] * d / jnp.sqrt(v + 1e-5) + be_ref[...]
    if not last:
        y = jnp.maximum(y, 0.0)
    o_ref[...] = y + x


@functools.partial(jax.jit, static_argnames=("last",))
def _tc_layer(agg2, x, p, last):
    return pl.pallas_call(
        functools.partial(_tc_layer_kernel, last=last),
        out_shape=jax.ShapeDtypeStruct((N_NODES, HIDDEN), jnp.float32),
    )(agg2, x, p["W1"], p["b1"].reshape(1, HIDDEN), p["W2"],
      p["b2"].reshape(1, HIDDEN), p["gamma"].reshape(1, HIDDEN),
      p["beta"].reshape(1, HIDDEN))


def kernel(z, edge_index, edge_attr, params):
    src3 = edge_index[0].astype(jnp.int32).reshape(NW * NCHUNK, 1, CHUNK)
    dst3 = edge_index[1].astype(jnp.int32).reshape(NW * NCHUNK, 1, CHUNK)
    x = _tc_emb(z.astype(jnp.int32), params["emb"])
    for i, p in enumerate(params["layers"]):
        agg2 = _sc_agg(x, edge_attr, src3, dst3)
        x = _tc_layer(agg2, x, p, last=(i == NUM_CONVS - 1))
    return x


# speedup vs baseline: 1.0822x; 1.0033x over previous
import functools

import jax
import jax.numpy as jnp
from jax import lax
from jax.experimental import pallas as pl
from jax.experimental.pallas import tpu as pltpu
from jax.experimental.pallas import tpu_sc as plsc

N_NODES = 10000
HIDDEN = 128
N_EDGES = 320000
NUM_CONVS = 3

NC = 2
NS = 16
NW = NC * NS
E_PER_W = N_EDGES // NW
CHUNK = 40
NCHUNK = E_PER_W // CHUNK
NPAIR = NCHUNK // 2
RCHUNK = 400
NRCHUNK = N_NODES // RCHUNK
HV = HIDDEN // 16


def _sc_agg_kernel(x_hbm, ea_hbm, src_hbm, dst_hbm, out_hbm,
                   sidx, didx, ea_v, row_v, sc_v, agg_sh,
                   isem0, isem1, dsem0, dsem1, gesem0, gesem1, ssem0, ssem1):
    c = lax.axis_index("c")
    s = lax.axis_index("s")
    wid = c * NS + s
    base_ch = wid * NCHUNK
    isem = (isem0, isem1)
    dsem = (dsem0, dsem1)
    gesem = (gesem0, gesem1)
    ssem = (ssem0, ssem1)

    def zbody(i, _):
        for k in range(HV):
            row_v[0, i, pl.ds(k * 16, 16)] = jnp.zeros((16,), jnp.float32)
        return 0
    lax.fori_loop(0, CHUNK, zbody, 0)

    def zcopy(r, _):
        idx = s + r * NS

        @pl.when(idx < NCHUNK)
        def _():
            pltpu.sync_copy(row_v.at[0], agg_sh.at[pl.ds(idx * CHUNK, CHUNK)])
        return 0
    lax.fori_loop(0, (NCHUNK + NS - 1) // NS, zcopy, 0)

    plsc.subcore_barrier()

    def _issue_gather(j, b):
        pltpu.async_copy(x_hbm.at[sidx.at[b, 0]], row_v.at[b], gesem[b])
        pltpu.async_copy(ea_hbm.at[pl.ds(wid * E_PER_W + j * CHUNK, CHUNK)],
                         ea_v.at[b], gesem[b])

    def _wait_ge(b):
        pltpu.make_async_copy(x_hbm.at[pl.ds(0, CHUNK)], row_v.at[b],
                              gesem[b]).wait()
        pltpu.make_async_copy(ea_hbm.at[pl.ds(0, CHUNK)], ea_v.at[b],
                              gesem[b]).wait()

    pltpu.async_copy(src_hbm.at[base_ch], sidx.at[0], isem0)
    pltpu.async_copy(src_hbm.at[base_ch + 1], sidx.at[1], isem1)
    pltpu.make_async_copy(src_hbm.at[0], sidx.at[0], isem0).wait()
    _issue_gather(0, 0)

    def body(t, _):
        for b in range(2):
            j = 2 * t + b

            if b == 0:
                pltpu.make_async_copy(src_hbm.at[0], sidx.at[1],
                                      isem[1]).wait()
                _issue_gather(j + 1, 1)
            else:
                @pl.when(t < NPAIR - 1)
                def _():
                    pltpu.make_async_copy(src_hbm.at[0], sidx.at[0],
                                          isem[0]).wait()
                    _issue_gather(j + 1, 0)

            _wait_ge(b)

            @pl.when(t >= 1)
            def _():
                pltpu.make_async_copy(sc_v.at[b], agg_sh.at[pl.ds(0, CHUNK)],
                                      ssem[b]).wait()

            @pl.when(t < NPAIR - 1)
            def _():
                pltpu.async_copy(src_hbm.at[base_ch + j + 2], sidx.at[b],
                                 isem[b])

            pltpu.async_copy(dst_hbm.at[base_ch + j], didx.at[b], dsem[b])

            def rbody(i, _):
                for k in range(HV):
                    a = row_v[b, i, pl.ds(k * 16, 16)]
                    e = ea_v[b, i, pl.ds(k * 16, 16)]
                    sc_v[b, i, pl.ds(k * 16, 16)] = jnp.maximum(a + e, 0.0)
                return 0
            lax.fori_loop(0, CHUNK, rbody, 0)

            pltpu.make_async_copy(dst_hbm.at[0], didx.at[b], dsem[b]).wait()
            pltpu.async_copy(sc_v.at[b], agg_sh.at[didx.at[b, 0]], ssem[b],
                             add=True)
        return 0
    lax.fori_loop(0, NPAIR, body, 0)

    for b in range(2):
        pltpu.make_async_copy(sc_v.at[b], agg_sh.at[pl.ds(0, CHUNK)],
                              ssem[b]).wait()

    plsc.subcore_barrier()

    def ocopy(r, _):
        idx = s + r * NS

        @pl.when(idx < NRCHUNK)
        def _():
            pltpu.sync_copy(agg_sh.at[pl.ds(idx * RCHUNK, RCHUNK)],
                            out_hbm.at[c, pl.ds(idx * RCHUNK, RCHUNK)])
        return 0
    lax.fori_loop(0, (NRCHUNK + NS - 1) // NS, ocopy, 0)


@jax.jit
def _sc_agg(x, edge_attr, src3, dst3):
    mesh = plsc.VectorSubcoreMesh(core_axis_name="c", subcore_axis_name="s")
    return pl.kernel(
        _sc_agg_kernel,
        out_type=jax.ShapeDtypeStruct((NC, N_NODES, HIDDEN), jnp.float32),
        mesh=mesh,
        scratch_types=[
            pltpu.VMEM((2, 1, CHUNK), jnp.int32),
            pltpu.VMEM((2, 1, CHUNK), jnp.int32),
            pltpu.VMEM((2, CHUNK, HIDDEN), jnp.float32),
            pltpu.VMEM((2, CHUNK, HIDDEN), jnp.float32),
            pltpu.VMEM((2, CHUNK, HIDDEN), jnp.float32),
            pltpu.VMEM_SHARED((N_NODES, HIDDEN), jnp.float32),
            pltpu.SemaphoreType.DMA,
            pltpu.SemaphoreType.DMA,
            pltpu.SemaphoreType.DMA,
            pltpu.SemaphoreType.DMA,
            pltpu.SemaphoreType.DMA,
            pltpu.SemaphoreType.DMA,
            pltpu.SemaphoreType.DMA,
            pltpu.SemaphoreType.DMA,
        ],
    )(x, edge_attr, src3, dst3)


def _tc_emb_kernel(z_ref, emb_ref, x_ref):
    z = z_ref[...]
    cols = lax.broadcasted_iota(jnp.int32, (N_NODES, 100), 1)
    oh = jnp.where(z == cols, 1.0, 0.0).astype(jnp.float32)
    x_ref[...] = jnp.dot(oh, emb_ref[...], preferred_element_type=jnp.float32,
                         precision=lax.Precision.HIGHEST)


@jax.jit
def _tc_emb(z, emb):
    return pl.pallas_call(
        _tc_emb_kernel,
        out_shape=jax.ShapeDtypeStruct((N_NODES, HIDDEN), jnp.float32),
    )(z.reshape(N_NODES, 1), emb)


def _tc_layer_kernel(agg_ref, x_ref, w1_ref, b1_ref, w2_ref, b2_ref,
                     g_ref, be_ref, o_ref, *, last):
    x = x_ref[...]
    out = agg_ref[0] + agg_ref[1] + x
    h = jnp.maximum(jnp.dot(out, w1_ref[...],
                            preferred_element_type=jnp.float32) + b1_ref[...], 0.0)
    y = jnp.dot(h, w2_ref[...], preferred_element_type=jnp.float32) + b2_ref[...]
    m = jnp.mean(y, axis=0, keepdims=True)
    d = y - m
    v = jnp.mean(d * d, axis=0, keepdims=True)
    y = g_ref[...] * d / jnp.sqrt(v + 1e-5) + be_ref[...]
    if not last:
        y = jnp.maximum(y, 0.0)
    o_ref[...] = y + x


@functools.partial(jax.jit, static_argnames=("last",))
def _tc_layer(agg2, x, p, last):
    return pl.pallas_call(
        functools.partial(_tc_layer_kernel, last=last),
        out_shape=jax.ShapeDtypeStruct((N_NODES, HIDDEN), jnp.float32),
    )(agg2, x, p["W1"], p["b1"].reshape(1, HIDDEN), p["W2"],
      p["b2"].reshape(1, HIDDEN), p["gamma"].reshape(1, HIDDEN),
      p["beta"].reshape(1, HIDDEN))


def kernel(z, edge_index, edge_attr, params):
    src3 = edge_index[0].astype(jnp.int32).reshape(NW * NCHUNK, 1, CHUNK)
    dst3 = edge_index[1].astype(jnp.int32).reshape(NW * NCHUNK, 1, CHUNK)
    x = _tc_emb(z.astype(jnp.int32), params["emb"])
    for i, p in enumerate(params["layers"]):
        agg2 = _sc_agg(x, edge_attr, src3, dst3)
        x = _tc_layer(agg2, x, p, last=(i == NUM_CONVS - 1))
    return x
